# baseline (speedup 1.0000x reference)
import jax
import jax.numpy as jnp
from jax.experimental import pallas as pl
from jax.experimental.pallas import tpu as pltpu

_BM = 512


def _bf16(v):
    return v.astype(jnp.bfloat16)


def _gcn_body(x_ref, w1_ref, b1_ref, w2_ref, b2_ref, adj_a_ref, adj_b_ref,
              o_ref, s_s, h_s):
    l = pl.program_id(0)
    i = pl.program_id(1)
    bm2 = _BM // 2

    @pl.when((l == 0) & (i == 0))
    def _():
        s_s[...] = _bf16(jax.lax.dot(
            _bf16(x_ref[...]), _bf16(w1_ref[...]),
            preferred_element_type=jnp.float32))

    @pl.when((l == 1) & (i == 0))
    def _():
        s_s[...] = _bf16(jax.lax.dot(
            _bf16(h_s[...]), _bf16(w2_ref[...]),
            preferred_element_type=jnp.float32))

    b = jnp.where(l == 0, b1_ref[...], b2_ref[...])
    s = s_s[...]
    ya = jax.lax.dot(_bf16(adj_a_ref[...]), s,
                     preferred_element_type=jnp.float32)
    ya = ya + b
    ya = jnp.where(ya >= 0, ya, 0.01 * ya)
    yb = jax.lax.dot(_bf16(adj_b_ref[...]), s,
                     preferred_element_type=jnp.float32)
    yb = yb + b
    yb = jnp.where(yb >= 0, yb, 0.01 * yb)

    @pl.when(l == 0)
    def _():
        h_s[pl.ds(i * _BM, bm2), :] = ya
        h_s[pl.ds(i * _BM + bm2, bm2), :] = yb

    @pl.when(l == 1)
    def _():
        o_ref[pl.ds(0, bm2), :] = ya
        o_ref[pl.ds(bm2, bm2), :] = yb


def kernel(x, adj, W1, b1, W2, b2):
    n, d_in = x.shape
    d_hid = W1.shape[1]
    d_out = W2.shape[1]
    nb = n // _BM
    return pl.pallas_call(
        _gcn_body,
        grid=(2, nb),
        in_specs=[
            pl.BlockSpec((n, d_in), lambda l, i: (0, 0)),
            pl.BlockSpec((d_in, d_hid), lambda l, i: (0, 0)),
            pl.BlockSpec((1, d_hid), lambda l, i: (0, 0)),
            pl.BlockSpec((d_hid, d_out), lambda l, i: (0, 0)),
            pl.BlockSpec((1, d_out), lambda l, i: (0, 0)),
            pl.BlockSpec((_BM // 2, n), lambda l, i: (2 * i, 0)),
            pl.BlockSpec((_BM // 2, n), lambda l, i: (2 * i + 1, 0)),
        ],
        out_specs=pl.BlockSpec((_BM, d_out), lambda l, i: (l * i, 0)),
        out_shape=jax.ShapeDtypeStruct((n, d_out), jnp.float32),
        scratch_shapes=[
            pltpu.VMEM((n, d_hid), jnp.bfloat16),
            pltpu.VMEM((n, d_hid), jnp.float32),
        ],
        compiler_params=pltpu.CompilerParams(
            dimension_semantics=("arbitrary", "arbitrary"),
        ),
    )(x, W1, b1.reshape(1, d_hid), W2, b2.reshape(1, d_out), adj, adj)
